# Initial kernel scaffold; baseline (speedup 1.0000x reference)
import functools

import jax
import jax.numpy as jnp
from jax import lax
from jax.experimental import pallas as pl
from jax.experimental.pallas import tpu as pltpu
from jax.experimental.pallas import tpu_sc as plsc

N = 10000
D = 128
E = 320000

NC = 2
NS = 16
NW = NC * NS
CH = 128
NPAD = 10240
EW = 10240
NCHUNK = EW // CH
EPAD = NW * EW
RPW = NPAD // NS

R = 640
G = NPAD // R
EPS = 1e-5


_sc_cache = {}


def _zero_fill(ref, rows, cols):
    zv = jnp.zeros((16,), jnp.float32)

    def body(i, _):
        for k in range(cols // 16):
            ref[i, pl.ds(k * 16, 16)] = zv
        return 0

    lax.fori_loop(0, rows, body, 0)


def _get_deg_call():
    if "deg" in _sc_cache:
        return _sc_cache["deg"]
    mesh = plsc.VectorSubcoreMesh(core_axis_name="c", subcore_axis_name="s")

    @functools.partial(
        pl.kernel,
        mesh=mesh,
        out_type=jax.ShapeDtypeStruct((NC, NPAD, 16), jnp.float32),
        scratch_types=[
            pltpu.VMEM((NCHUNK, CH), jnp.int32),
            pltpu.VMEM((CH, 16), jnp.float32),
            pltpu.VMEM((CH, 16), jnp.float32),
            pltpu.VMEM_SHARED((NPAD, 16), jnp.float32),
            pltpu.SemaphoreType.DMA,
        ],
    )
    def deg_kernel(dst_hbm, out_hbm, idx_v, ones_v, zbuf, deg_sh, sem):
        c = lax.axis_index("c")
        s = lax.axis_index("s")
        wid = s * NC + c

        ov = jnp.full((16,), 1.0, jnp.float32)

        def fill(i, _):
            ones_v[i, :] = ov
            return 0

        lax.fori_loop(0, CH, fill, 0)
        _zero_fill(zbuf, CH, 16)

        pltpu.sync_copy(dst_hbm.at[wid], idx_v)

        for k in range(RPW // CH):
            pltpu.sync_copy(zbuf, deg_sh.at[pl.ds(s * RPW + k * CH, CH)])
        plsc.subcore_barrier()

        def body(j, _):
            pltpu.sync_copy(ones_v, deg_sh.at[idx_v.at[j]], add=True)
            return 0

        lax.fori_loop(0, NCHUNK, body, 0)
        plsc.subcore_barrier()

        pltpu.sync_copy(
            deg_sh.at[pl.ds(s * RPW, RPW)],
            out_hbm.at[c, pl.ds(s * RPW, RPW)],
        )

    _sc_cache["deg"] = deg_kernel
    return deg_kernel


def _get_edge_call():
    if "edge" in _sc_cache:
        return _sc_cache["edge"]
    mesh = plsc.VectorSubcoreMesh(core_axis_name="c", subcore_axis_name="s")

    @functools.partial(
        pl.kernel,
        mesh=mesh,
        out_type=jax.ShapeDtypeStruct((NC, NPAD, D), jnp.float32),
        scratch_types=[
            pltpu.VMEM((NCHUNK, CH), jnp.int32),
            pltpu.VMEM((NCHUNK, CH), jnp.int32),
            pltpu.VMEM((CH, D), jnp.float32),
            pltpu.VMEM((CH, D), jnp.float32),
            pltpu.VMEM_SHARED((NPAD, D), jnp.float32),
            pltpu.SemaphoreType.DMA,
        ],
    )
    def edge_kernel(u_hbm, src_hbm, dst_hbm, out_hbm,
                    src_v, dst_v, rows_v, zbuf, acc_sh, sem):
        c = lax.axis_index("c")
        s = lax.axis_index("s")
        wid = s * NC + c

        _zero_fill(zbuf, CH, D)
        pltpu.sync_copy(src_hbm.at[wid], src_v)
        pltpu.sync_copy(dst_hbm.at[wid], dst_v)

        for k in range(RPW // CH):
            pltpu.sync_copy(zbuf, acc_sh.at[pl.ds(s * RPW + k * CH, CH)])
        plsc.subcore_barrier()

        def body(j, _):
            pltpu.async_copy(u_hbm.at[src_v.at[j]], rows_v, sem).wait()
            pltpu.sync_copy(rows_v, acc_sh.at[dst_v.at[j]], add=True)
            return 0

        lax.fori_loop(0, NCHUNK, body, 0)
        plsc.subcore_barrier()

        pltpu.sync_copy(
            acc_sh.at[pl.ds(s * RPW, RPW)],
            out_hbm.at[c, pl.ds(s * RPW, RPW)],
        )

    _sc_cache["edge"] = edge_kernel
    return edge_kernel


def _k1_body(degp_ref, x_ref, w_ref, u_ref, dinv_ref):
    deg = jnp.sum(degp_ref[...], axis=(0, 2)) * (1.0 / 16.0) + 1.0
    dinv = lax.rsqrt(deg)
    h = jnp.dot(x_ref[...], w_ref[...], preferred_element_type=jnp.float32)
    u_ref[...] = h * dinv[:, None]
    dinv_ref[...] = dinv[:, None]


_k1_call = pl.pallas_call(
    _k1_body,
    grid=(G,),
    in_specs=[
        pl.BlockSpec((NC, R, 16), lambda i: (0, i, 0)),
        pl.BlockSpec((R, D), lambda i: (i, 0)),
        pl.BlockSpec((D, D), lambda i: (0, 0)),
    ],
    out_specs=[
        pl.BlockSpec((R, D), lambda i: (i, 0)),
        pl.BlockSpec((R, 1), lambda i: (i, 0)),
    ],
    out_shape=[
        jax.ShapeDtypeStruct((NPAD, D), jnp.float32),
        jax.ShapeDtypeStruct((NPAD, 1), jnp.float32),
    ],
)


def _k3a_body(sacc_ref, u_ref, dinv_ref, b_ref, v_ref, stats_ref):
    i = pl.program_id(0)
    t = sacc_ref[0] + sacc_ref[1] + u_ref[...]
    v = t * dinv_ref[...] + b_ref[...]
    rows = i * R + lax.broadcasted_iota(jnp.int32, (R, 1), 0)
    vm = jnp.where(rows < N, v, 0.0)
    v_ref[...] = vm
    blk = jnp.concatenate(
        [jnp.sum(vm, axis=0, keepdims=True),
         jnp.sum(vm * vm, axis=0, keepdims=True)], axis=0)

    @pl.when(i == 0)
    def _():
        stats_ref[...] = blk

    @pl.when(i > 0)
    def _():
        stats_ref[...] += blk


_k3a_call = pl.pallas_call(
    _k3a_body,
    grid=(G,),
    in_specs=[
        pl.BlockSpec((NC, R, D), lambda i: (0, i, 0)),
        pl.BlockSpec((R, D), lambda i: (i, 0)),
        pl.BlockSpec((R, 1), lambda i: (i, 0)),
        pl.BlockSpec((1, D), lambda i: (0, 0)),
    ],
    out_specs=[
        pl.BlockSpec((R, D), lambda i: (i, 0)),
        pl.BlockSpec((2, D), lambda i: (0, 0)),
    ],
    out_shape=[
        jax.ShapeDtypeStruct((NPAD, D), jnp.float32),
        jax.ShapeDtypeStruct((2, D), jnp.float32),
    ],
)


def _bn_relu(v, stats, g, be, rows):
    mean = stats[0:1] * (1.0 / N)
    var = stats[1:2] * (1.0 / N) - mean * mean
    inv = lax.rsqrt(var + EPS)
    y = jnp.maximum((v - mean) * (inv * g) + be, 0.0)
    return jnp.where(rows < N, y, 0.0)


def _k3b_body(v_ref, stats_ref, g_ref, be_ref, w_ref, dinv_ref, u2_ref):
    i = pl.program_id(0)
    rows = i * R + lax.broadcasted_iota(jnp.int32, (R, 1), 0)
    y = _bn_relu(v_ref[...], stats_ref[...], g_ref[...], be_ref[...], rows)
    u2_ref[...] = jnp.dot(
        y, w_ref[...], preferred_element_type=jnp.float32) * dinv_ref[...]


_k3b_call = pl.pallas_call(
    _k3b_body,
    grid=(G,),
    in_specs=[
        pl.BlockSpec((R, D), lambda i: (i, 0)),
        pl.BlockSpec((2, D), lambda i: (0, 0)),
        pl.BlockSpec((1, D), lambda i: (0, 0)),
        pl.BlockSpec((1, D), lambda i: (0, 0)),
        pl.BlockSpec((D, D), lambda i: (0, 0)),
        pl.BlockSpec((R, 1), lambda i: (i, 0)),
    ],
    out_specs=pl.BlockSpec((R, D), lambda i: (i, 0)),
    out_shape=jax.ShapeDtypeStruct((NPAD, D), jnp.float32),
)


def _k5b_body(v_ref, stats_ref, g_ref, be_ref, out_ref):
    i = pl.program_id(0)
    rows = i * R + lax.broadcasted_iota(jnp.int32, (R, 1), 0)
    out_ref[...] = _bn_relu(
        v_ref[...], stats_ref[...], g_ref[...], be_ref[...], rows)


_k5b_call = pl.pallas_call(
    _k5b_body,
    grid=(G,),
    in_specs=[
        pl.BlockSpec((R, D), lambda i: (i, 0)),
        pl.BlockSpec((2, D), lambda i: (0, 0)),
        pl.BlockSpec((1, D), lambda i: (0, 0)),
        pl.BlockSpec((1, D), lambda i: (0, 0)),
    ],
    out_specs=pl.BlockSpec((R, D), lambda i: (i, 0)),
    out_shape=jax.ShapeDtypeStruct((NPAD, D), jnp.float32),
)


def kernel(x, edge_index, W1, b1, gamma1, beta1, W2, b2, gamma2, beta2):
    xp = jnp.zeros((NPAD, D), jnp.float32).at[:N].set(x)
    ei = edge_index.astype(jnp.int32)
    pad = jnp.full((2, EPAD - E), NPAD - 1, jnp.int32)
    eip = jnp.concatenate([ei, pad], axis=1)
    srcp = eip[0].reshape(NW, NCHUNK, CH)
    dstp = eip[1].reshape(NW, NCHUNK, CH)

    degp = _get_deg_call()(dstp)
    u1, dinv = _k1_call(degp, xp, W1)

    edge_call = _get_edge_call()
    sacc1 = edge_call(u1, srcp, dstp)
    v1, st1 = _k3a_call(sacc1, u1, dinv, b1.reshape(1, D))
    u2 = _k3b_call(v1, st1, gamma1.reshape(1, D), beta1.reshape(1, D), W2, dinv)

    sacc2 = edge_call(u2, srcp, dstp)
    v2, st2 = _k3a_call(sacc2, u2, dinv, b2.reshape(1, D))
    out = _k5b_call(v2, st2, gamma2.reshape(1, D), beta2.reshape(1, D))
    return out[:N]


# speedup vs baseline: 8.0038x; 8.0038x over previous
import functools

import jax
import jax.numpy as jnp
from jax import lax
from jax.experimental import pallas as pl
from jax.experimental.pallas import tpu as pltpu
from jax.experimental.pallas import tpu_sc as plsc

N = 10000
D = 128
E = 320000

NC = 2
NS = 16
NW = NC * NS
CH = 128
NPAD = 10240
EW = 10240
NCHUNK = EW // CH
EPAD = NW * EW
RPW = NPAD // NS

R = 640
G = NPAD // R
EPS = 1e-5


_sc_cache = {}


def _zero_fill(ref, rows, cols):
    zv = jnp.zeros((16,), jnp.float32)

    def body(i, _):
        for k in range(cols // 16):
            ref[i, pl.ds(k * 16, 16)] = zv
        return 0

    lax.fori_loop(0, rows, body, 0)


def _get_deg_call():
    if "deg" in _sc_cache:
        return _sc_cache["deg"]
    mesh = plsc.VectorSubcoreMesh(core_axis_name="c", subcore_axis_name="s")

    @functools.partial(
        pl.kernel,
        mesh=mesh,
        out_type=jax.ShapeDtypeStruct((NC, NPAD, 16), jnp.float32),
        scratch_types=[
            pltpu.VMEM((NCHUNK, CH), jnp.int32),
            pltpu.VMEM((CH, 16), jnp.float32),
            pltpu.VMEM((CH, 16), jnp.float32),
            pltpu.VMEM_SHARED((NPAD, 16), jnp.float32),
            pltpu.SemaphoreType.DMA,
        ],
    )
    def deg_kernel(dst_hbm, out_hbm, idx_v, ones_v, zbuf, deg_sh, sem):
        c = lax.axis_index("c")
        s = lax.axis_index("s")
        wid = s * NC + c

        ov = jnp.full((16,), 1.0, jnp.float32)

        def fill(i, _):
            ones_v[i, :] = ov
            return 0

        lax.fori_loop(0, CH, fill, 0)
        _zero_fill(zbuf, CH, 16)

        pltpu.sync_copy(dst_hbm.at[wid], idx_v)

        for k in range(RPW // CH):
            pltpu.sync_copy(zbuf, deg_sh.at[pl.ds(s * RPW + k * CH, CH)])
        plsc.subcore_barrier()

        def body(j, _):
            pltpu.sync_copy(ones_v, deg_sh.at[idx_v.at[j]], add=True)
            return 0

        lax.fori_loop(0, NCHUNK, body, 0)
        plsc.subcore_barrier()

        pltpu.sync_copy(
            deg_sh.at[pl.ds(s * RPW, RPW)],
            out_hbm.at[c, pl.ds(s * RPW, RPW)],
        )

    _sc_cache["deg"] = deg_kernel
    return deg_kernel


def _get_edge_call():
    if "edge" in _sc_cache:
        return _sc_cache["edge"]
    mesh = plsc.VectorSubcoreMesh(core_axis_name="c", subcore_axis_name="s")

    @functools.partial(
        pl.kernel,
        mesh=mesh,
        out_type=jax.ShapeDtypeStruct((NC, NPAD, D), jnp.float32),
        scratch_types=[
            pltpu.VMEM((NCHUNK, CH), jnp.int32),
            pltpu.VMEM((NCHUNK, CH), jnp.int32),
            pltpu.VMEM((CH, D), jnp.float32),
            pltpu.VMEM_SHARED((NPAD, D), jnp.float32),
            pltpu.SemaphoreType.DMA,
        ],
    )
    def edge_kernel(u_hbm, src_hbm, dst_hbm, out_hbm,
                    src_v, dst_v, rows_v, acc_sh, sem):
        c = lax.axis_index("c")
        s = lax.axis_index("s")
        wid = s * NC + c

        _zero_fill(rows_v, CH, D)
        pltpu.sync_copy(src_hbm.at[wid], src_v)
        pltpu.sync_copy(dst_hbm.at[wid], dst_v)

        for k in range(RPW // CH):
            pltpu.sync_copy(rows_v, acc_sh.at[pl.ds(s * RPW + k * CH, CH)])
        plsc.subcore_barrier()

        def body(j, _):
            pltpu.async_copy(u_hbm.at[src_v.at[j]], rows_v, sem).wait()
            pltpu.sync_copy(rows_v, acc_sh.at[dst_v.at[j]], add=True)
            return 0

        lax.fori_loop(0, NCHUNK, body, 0)
        plsc.subcore_barrier()

        pltpu.sync_copy(
            acc_sh.at[pl.ds(s * RPW, RPW)],
            out_hbm.at[c, pl.ds(s * RPW, RPW)],
        )

    _sc_cache["edge"] = edge_kernel
    return edge_kernel


def _k1_body(degp_ref, x_ref, w_ref, u_ref, dinv_ref):
    deg = jnp.sum(degp_ref[...], axis=(0, 2)) * (1.0 / 16.0) + 1.0
    dinv = lax.rsqrt(deg)
    h = jnp.dot(x_ref[...], w_ref[...], preferred_element_type=jnp.float32)
    u_ref[...] = h * dinv[:, None]
    dinv_ref[...] = dinv[:, None]


_k1_call = pl.pallas_call(
    _k1_body,
    grid=(G,),
    in_specs=[
        pl.BlockSpec((NC, R, 16), lambda i: (0, i, 0)),
        pl.BlockSpec((R, D), lambda i: (i, 0)),
        pl.BlockSpec((D, D), lambda i: (0, 0)),
    ],
    out_specs=[
        pl.BlockSpec((R, D), lambda i: (i, 0)),
        pl.BlockSpec((R, 1), lambda i: (i, 0)),
    ],
    out_shape=[
        jax.ShapeDtypeStruct((NPAD, D), jnp.float32),
        jax.ShapeDtypeStruct((NPAD, 1), jnp.float32),
    ],
)


def _k3a_body(sacc_ref, u_ref, dinv_ref, b_ref, v_ref, stats_ref):
    i = pl.program_id(0)
    t = sacc_ref[0] + sacc_ref[1] + u_ref[...]
    v = t * dinv_ref[...] + b_ref[...]
    rows = i * R + lax.broadcasted_iota(jnp.int32, (R, 1), 0)
    vm = jnp.where(rows < N, v, 0.0)
    v_ref[...] = vm
    blk = jnp.concatenate(
        [jnp.sum(vm, axis=0, keepdims=True),
         jnp.sum(vm * vm, axis=0, keepdims=True)], axis=0)

    @pl.when(i == 0)
    def _():
        stats_ref[...] = blk

    @pl.when(i > 0)
    def _():
        stats_ref[...] += blk


_k3a_call = pl.pallas_call(
    _k3a_body,
    grid=(G,),
    in_specs=[
        pl.BlockSpec((NC, R, D), lambda i: (0, i, 0)),
        pl.BlockSpec((R, D), lambda i: (i, 0)),
        pl.BlockSpec((R, 1), lambda i: (i, 0)),
        pl.BlockSpec((1, D), lambda i: (0, 0)),
    ],
    out_specs=[
        pl.BlockSpec((R, D), lambda i: (i, 0)),
        pl.BlockSpec((2, D), lambda i: (0, 0)),
    ],
    out_shape=[
        jax.ShapeDtypeStruct((NPAD, D), jnp.float32),
        jax.ShapeDtypeStruct((2, D), jnp.float32),
    ],
)


def _bn_relu(v, stats, g, be, rows):
    mean = stats[0:1] * (1.0 / N)
    var = stats[1:2] * (1.0 / N) - mean * mean
    inv = lax.rsqrt(var + EPS)
    y = jnp.maximum((v - mean) * (inv * g) + be, 0.0)
    return jnp.where(rows < N, y, 0.0)


def _k3b_body(v_ref, stats_ref, g_ref, be_ref, w_ref, dinv_ref, u2_ref):
    i = pl.program_id(0)
    rows = i * R + lax.broadcasted_iota(jnp.int32, (R, 1), 0)
    y = _bn_relu(v_ref[...], stats_ref[...], g_ref[...], be_ref[...], rows)
    u2_ref[...] = jnp.dot(
        y, w_ref[...], preferred_element_type=jnp.float32) * dinv_ref[...]


_k3b_call = pl.pallas_call(
    _k3b_body,
    grid=(G,),
    in_specs=[
        pl.BlockSpec((R, D), lambda i: (i, 0)),
        pl.BlockSpec((2, D), lambda i: (0, 0)),
        pl.BlockSpec((1, D), lambda i: (0, 0)),
        pl.BlockSpec((1, D), lambda i: (0, 0)),
        pl.BlockSpec((D, D), lambda i: (0, 0)),
        pl.BlockSpec((R, 1), lambda i: (i, 0)),
    ],
    out_specs=pl.BlockSpec((R, D), lambda i: (i, 0)),
    out_shape=jax.ShapeDtypeStruct((NPAD, D), jnp.float32),
)


def _k5b_body(v_ref, stats_ref, g_ref, be_ref, out_ref):
    i = pl.program_id(0)
    rows = i * R + lax.broadcasted_iota(jnp.int32, (R, 1), 0)
    out_ref[...] = _bn_relu(
        v_ref[...], stats_ref[...], g_ref[...], be_ref[...], rows)


_k5b_call = pl.pallas_call(
    _k5b_body,
    grid=(G,),
    in_specs=[
        pl.BlockSpec((R, D), lambda i: (i, 0)),
        pl.BlockSpec((2, D), lambda i: (0, 0)),
        pl.BlockSpec((1, D), lambda i: (0, 0)),
        pl.BlockSpec((1, D), lambda i: (0, 0)),
    ],
    out_specs=pl.BlockSpec((R, D), lambda i: (i, 0)),
    out_shape=jax.ShapeDtypeStruct((NPAD, D), jnp.float32),
)


def kernel(x, edge_index, W1, b1, gamma1, beta1, W2, b2, gamma2, beta2):
    xp = jnp.zeros((NPAD, D), jnp.float32).at[:N].set(x)
    ei = edge_index.astype(jnp.int32)
    pad = jnp.full((2, EPAD - E), NPAD - 1, jnp.int32)
    eip = jnp.concatenate([ei, pad], axis=1)
    srcp = eip[0].reshape(NW, NCHUNK, CH)
    dstp = eip[1].reshape(NW, NCHUNK, CH)

    degp = _get_deg_call()(dstp)
    u1, dinv = _k1_call(degp, xp, W1)

    edge_call = _get_edge_call()
    sacc1 = edge_call(u1, srcp, dstp)
    v1, st1 = _k3a_call(sacc1, u1, dinv, b1.reshape(1, D))
    u2 = _k3b_call(v1, st1, gamma1.reshape(1, D), beta1.reshape(1, D), W2, dinv)

    sacc2 = edge_call(u2, srcp, dstp)
    v2, st2 = _k3a_call(sacc2, u2, dinv, b2.reshape(1, D))
    out = _k5b_call(v2, st2, gamma2.reshape(1, D), beta2.reshape(1, D))
    return out[:N]


# speedup vs baseline: 8.8350x; 1.1039x over previous
import functools

import jax
import jax.numpy as jnp
from jax import lax
from jax.experimental import pallas as pl
from jax.experimental.pallas import tpu as pltpu
from jax.experimental.pallas import tpu_sc as plsc

N = 10000
D = 128
E = 320000

NC = 2
NS = 16
NW = NC * NS
CH = 128
BCH = 40
NPAD = 10240
EW = 10240
NCHUNK = EW // CH
EPAD = NW * EW
RPW = NPAD // NS

R = 640
G = NPAD // R
EPS = 1e-5


_sc_cache = {}


def _zero_fill(ref, rows, cols):
    zv = jnp.zeros((16,), jnp.float32)

    def body(i, _):
        for k in range(cols // 16):
            ref[i, pl.ds(k * 16, 16)] = zv
        return 0

    lax.fori_loop(0, rows, body, 0)


def _get_deg_call():
    if "deg" in _sc_cache:
        return _sc_cache["deg"]
    mesh = plsc.VectorSubcoreMesh(core_axis_name="c", subcore_axis_name="s")

    @functools.partial(
        pl.kernel,
        mesh=mesh,
        out_type=jax.ShapeDtypeStruct((NC, NPAD, 16), jnp.float32),
        scratch_types=[
            pltpu.VMEM((NCHUNK, CH), jnp.int32),
            pltpu.VMEM((CH, 16), jnp.float32),
            pltpu.VMEM((CH, 16), jnp.float32),
            pltpu.VMEM_SHARED((NPAD, 16), jnp.float32),
            pltpu.SemaphoreType.DMA,
        ],
    )
    def deg_kernel(dst_hbm, out_hbm, idx_v, ones_v, zbuf, deg_sh, sem):
        c = lax.axis_index("c")
        s = lax.axis_index("s")
        wid = s * NC + c

        ov = jnp.full((16,), 1.0, jnp.float32)

        def fill(i, _):
            ones_v[i, :] = ov
            return 0

        lax.fori_loop(0, CH, fill, 0)
        _zero_fill(zbuf, CH, 16)

        pltpu.sync_copy(dst_hbm.at[wid], idx_v)

        for k in range(RPW // CH):
            pltpu.sync_copy(zbuf, deg_sh.at[pl.ds(s * RPW + k * CH, CH)])
        plsc.subcore_barrier()

        def body(j, _):
            pltpu.sync_copy(ones_v, deg_sh.at[idx_v.at[j]], add=True)
            return 0

        lax.fori_loop(0, NCHUNK, body, 0)
        plsc.subcore_barrier()

        pltpu.sync_copy(
            deg_sh.at[pl.ds(s * RPW, RPW)],
            out_hbm.at[c, pl.ds(s * RPW, RPW)],
        )

    _sc_cache["deg"] = deg_kernel
    return deg_kernel


def _get_edge_call():
    if "edge" in _sc_cache:
        return _sc_cache["edge"]
    mesh = plsc.VectorSubcoreMesh(core_axis_name="c", subcore_axis_name="s")

    @functools.partial(
        pl.kernel,
        mesh=mesh,
        out_type=jax.ShapeDtypeStruct((NC, NPAD, D), jnp.float32),
        scratch_types=[
            pltpu.VMEM((BCH, CH), jnp.int32),
            pltpu.VMEM((BCH, CH), jnp.int32),
            pltpu.VMEM((CH, D), jnp.float32),
            pltpu.VMEM((CH, D), jnp.float32),
            pltpu.VMEM_SHARED((NPAD, D), jnp.float32),
            pltpu.SemaphoreType.DMA,
            pltpu.SemaphoreType.DMA,
            pltpu.SemaphoreType.DMA,
            pltpu.SemaphoreType.DMA,
        ],
    )
    def edge_kernel(u_hbm, src_hbm, dst_hbm, out_hbm,
                    src_v, dst_v, r0, r1, acc_sh, sg0, sg1, ss0, ss1):
        c = lax.axis_index("c")
        s = lax.axis_index("s")
        wid = s * NC + c

        _zero_fill(r0, CH, D)

        for k in range(RPW // CH):
            pltpu.sync_copy(r0, acc_sh.at[pl.ds(s * RPW + k * CH, CH)])
        plsc.subcore_barrier()

        def gather(j, buf, sem):
            pltpu.async_copy(u_hbm.at[src_v.at[j]], buf, sem)

        def gather_wait(buf, sem):
            pltpu.make_async_copy(u_hbm.at[src_v.at[0]], buf, sem).wait()

        def scatter(j, buf, sem):
            pltpu.async_copy(buf, acc_sh.at[dst_v.at[j]], sem, add=True)

        def scatter_wait(j, buf, sem):
            pltpu.make_async_copy(buf, acc_sh.at[dst_v.at[j]], sem).wait()

        for blk in range(NCHUNK // BCH):
            pltpu.sync_copy(src_hbm.at[wid, pl.ds(blk * BCH, BCH)], src_v)
            pltpu.sync_copy(dst_hbm.at[wid, pl.ds(blk * BCH, BCH)], dst_v)

            gather(0, r0, sg0)

            def body(t, _):
                j = 2 * t
                gather(j + 1, r1, sg1)
                gather_wait(r0, sg0)
                pltpu.sync_copy(r0, acc_sh.at[dst_v.at[j]], add=True)

                @pl.when(t + 1 < BCH // 2)
                def _():
                    gather(j + 2, r0, sg0)

                gather_wait(r1, sg1)
                pltpu.sync_copy(r1, acc_sh.at[dst_v.at[j + 1]], add=True)
                return 0

            lax.fori_loop(0, BCH // 2, body, 0)
        plsc.subcore_barrier()

        pltpu.sync_copy(
            acc_sh.at[pl.ds(s * RPW, RPW)],
            out_hbm.at[c, pl.ds(s * RPW, RPW)],
        )

    _sc_cache["edge"] = edge_kernel
    return edge_kernel


def _k1_body(degp_ref, x_ref, w_ref, u_ref, dinv_ref):
    deg = jnp.sum(degp_ref[...], axis=(0, 2)) * (1.0 / 16.0) + 1.0
    dinv = lax.rsqrt(deg)
    h = jnp.dot(x_ref[...], w_ref[...], preferred_element_type=jnp.float32)
    u_ref[...] = h * dinv[:, None]
    dinv_ref[...] = dinv[:, None]


_k1_call = pl.pallas_call(
    _k1_body,
    grid=(G,),
    in_specs=[
        pl.BlockSpec((NC, R, 16), lambda i: (0, i, 0)),
        pl.BlockSpec((R, D), lambda i: (i, 0)),
        pl.BlockSpec((D, D), lambda i: (0, 0)),
    ],
    out_specs=[
        pl.BlockSpec((R, D), lambda i: (i, 0)),
        pl.BlockSpec((R, 1), lambda i: (i, 0)),
    ],
    out_shape=[
        jax.ShapeDtypeStruct((NPAD, D), jnp.float32),
        jax.ShapeDtypeStruct((NPAD, 1), jnp.float32),
    ],
)


def _k3a_body(sacc_ref, u_ref, dinv_ref, b_ref, v_ref, stats_ref):
    i = pl.program_id(0)
    t = sacc_ref[0] + sacc_ref[1] + u_ref[...]
    v = t * dinv_ref[...] + b_ref[...]
    rows = i * R + lax.broadcasted_iota(jnp.int32, (R, 1), 0)
    vm = jnp.where(rows < N, v, 0.0)
    v_ref[...] = vm
    blk = jnp.concatenate(
        [jnp.sum(vm, axis=0, keepdims=True),
         jnp.sum(vm * vm, axis=0, keepdims=True)], axis=0)

    @pl.when(i == 0)
    def _():
        stats_ref[...] = blk

    @pl.when(i > 0)
    def _():
        stats_ref[...] += blk


_k3a_call = pl.pallas_call(
    _k3a_body,
    grid=(G,),
    in_specs=[
        pl.BlockSpec((NC, R, D), lambda i: (0, i, 0)),
        pl.BlockSpec((R, D), lambda i: (i, 0)),
        pl.BlockSpec((R, 1), lambda i: (i, 0)),
        pl.BlockSpec((1, D), lambda i: (0, 0)),
    ],
    out_specs=[
        pl.BlockSpec((R, D), lambda i: (i, 0)),
        pl.BlockSpec((2, D), lambda i: (0, 0)),
    ],
    out_shape=[
        jax.ShapeDtypeStruct((NPAD, D), jnp.float32),
        jax.ShapeDtypeStruct((2, D), jnp.float32),
    ],
)


def _bn_relu(v, stats, g, be, rows):
    mean = stats[0:1] * (1.0 / N)
    var = stats[1:2] * (1.0 / N) - mean * mean
    inv = lax.rsqrt(var + EPS)
    y = jnp.maximum((v - mean) * (inv * g) + be, 0.0)
    return jnp.where(rows < N, y, 0.0)


def _k3b_body(v_ref, stats_ref, g_ref, be_ref, w_ref, dinv_ref, u2_ref):
    i = pl.program_id(0)
    rows = i * R + lax.broadcasted_iota(jnp.int32, (R, 1), 0)
    y = _bn_relu(v_ref[...], stats_ref[...], g_ref[...], be_ref[...], rows)
    u2_ref[...] = jnp.dot(
        y, w_ref[...], preferred_element_type=jnp.float32) * dinv_ref[...]


_k3b_call = pl.pallas_call(
    _k3b_body,
    grid=(G,),
    in_specs=[
        pl.BlockSpec((R, D), lambda i: (i, 0)),
        pl.BlockSpec((2, D), lambda i: (0, 0)),
        pl.BlockSpec((1, D), lambda i: (0, 0)),
        pl.BlockSpec((1, D), lambda i: (0, 0)),
        pl.BlockSpec((D, D), lambda i: (0, 0)),
        pl.BlockSpec((R, 1), lambda i: (i, 0)),
    ],
    out_specs=pl.BlockSpec((R, D), lambda i: (i, 0)),
    out_shape=jax.ShapeDtypeStruct((NPAD, D), jnp.float32),
)


def _k5b_body(v_ref, stats_ref, g_ref, be_ref, out_ref):
    i = pl.program_id(0)
    rows = i * R + lax.broadcasted_iota(jnp.int32, (R, 1), 0)
    out_ref[...] = _bn_relu(
        v_ref[...], stats_ref[...], g_ref[...], be_ref[...], rows)


_k5b_call = pl.pallas_call(
    _k5b_body,
    grid=(G,),
    in_specs=[
        pl.BlockSpec((R, D), lambda i: (i, 0)),
        pl.BlockSpec((2, D), lambda i: (0, 0)),
        pl.BlockSpec((1, D), lambda i: (0, 0)),
        pl.BlockSpec((1, D), lambda i: (0, 0)),
    ],
    out_specs=pl.BlockSpec((R, D), lambda i: (i, 0)),
    out_shape=jax.ShapeDtypeStruct((NPAD, D), jnp.float32),
)


def kernel(x, edge_index, W1, b1, gamma1, beta1, W2, b2, gamma2, beta2):
    xp = jnp.zeros((NPAD, D), jnp.float32).at[:N].set(x)
    ei = edge_index.astype(jnp.int32)
    pad = jnp.full((2, EPAD - E), NPAD - 1, jnp.int32)
    eip = jnp.concatenate([ei, pad], axis=1)
    srcp = eip[0].reshape(NW, NCHUNK, CH)
    dstp = eip[1].reshape(NW, NCHUNK, CH)

    degp = _get_deg_call()(dstp)
    u1, dinv = _k1_call(degp, xp, W1)

    edge_call = _get_edge_call()
    sacc1 = edge_call(u1, srcp, dstp)
    v1, st1 = _k3a_call(sacc1, u1, dinv, b1.reshape(1, D))
    u2 = _k3b_call(v1, st1, gamma1.reshape(1, D), beta1.reshape(1, D), W2, dinv)

    sacc2 = edge_call(u2, srcp, dstp)
    v2, st2 = _k3a_call(sacc2, u2, dinv, b2.reshape(1, D))
    out = _k5b_call(v2, st2, gamma2.reshape(1, D), beta2.reshape(1, D))
    return out[:N]
